# baseline (speedup 1.0000x reference)
import functools

import jax
import jax.numpy as jnp
from jax import lax
from jax.experimental import pallas as pl
from jax.experimental.pallas import tpu as pltpu
from jax.experimental.pallas import tpu_sc as plsc

_LANES = 16
_CHUNK = 8


def kernel(nodes, neighbors, attention_scores, bias):
    del nodes
    N, K, _, D = neighbors.shape
    KD = K * D
    nbr = neighbors.reshape(N * KD)
    att = attention_scores.reshape(N * K)
    assert N % _CHUNK == 0 and D % _LANES == 0 and K % _LANES == 0
    n_chunks = N // _CHUNK
    num_workers = 32
    n_dblk = D // _LANES

    mesh = plsc.VectorSubcoreMesh(core_axis_name="c", subcore_axis_name="s")

    @functools.partial(
        pl.kernel,
        mesh=mesh,
        out_type=jax.ShapeDtypeStruct((N * D,), jnp.float32),
        scratch_types=[
            pltpu.VMEM((_CHUNK * KD,), jnp.float32),
            pltpu.VMEM((_CHUNK * KD,), jnp.float32),
            pltpu.VMEM((_CHUNK * K,), jnp.float32),
            pltpu.VMEM((_CHUNK * K,), jnp.float32),
            pltpu.VMEM((_CHUNK * D,), jnp.float32),
            pltpu.VMEM((D,), jnp.float32),
            pltpu.SemaphoreType.DMA,
            pltpu.SemaphoreType.DMA,
            pltpu.SemaphoreType.DMA,
            pltpu.SemaphoreType.DMA,
        ],
    )
    def sc_kernel(nbr_hbm, att_hbm, bias_hbm, out_hbm,
                  nbr_v0, nbr_v1, att_v0, att_v1, out_v, bias_v,
                  sn0, sn1, sa0, sa1):
        nbr_bufs = (nbr_v0, nbr_v1)
        att_bufs = (att_v0, att_v1)
        cid = lax.axis_index("c")
        sid = lax.axis_index("s")
        wid = sid * 2 + cid
        sems_n = (sn0, sn1)
        sems_a = (sa0, sa1)
        pltpu.sync_copy(bias_hbm, bias_v)
        n_my = (n_chunks - wid + num_workers - 1) // num_workers

        def chunk_base(t):
            return (wid + t * num_workers) * _CHUNK

        def issue(t, b):
            base = chunk_base(t)
            pltpu.async_copy(nbr_hbm.at[pl.ds(base * KD, _CHUNK * KD)],
                             nbr_bufs[b], sems_n[b])
            pltpu.async_copy(att_hbm.at[pl.ds(base * K, _CHUNK * K)],
                             att_bufs[b], sems_a[b])

        def drain(t, b):
            base = chunk_base(t)
            pltpu.make_async_copy(nbr_hbm.at[pl.ds(base * KD, _CHUNK * KD)],
                                  nbr_bufs[b], sems_n[b]).wait()
            pltpu.make_async_copy(att_hbm.at[pl.ds(base * K, _CHUNK * K)],
                                  att_bufs[b], sems_a[b]).wait()

        def compute(t, b):
            nv = nbr_bufs[b]
            av = att_bufs[b]

            for i in range(_CHUNK):
                nbase = i * KD
                abase = i * K
                obase = i * D
                accs = [bias_v[pl.ds(j * _LANES, _LANES)] for j in range(n_dblk)]
                att_rows = [
                    av[pl.ds(abase + kk * _LANES, _LANES)]
                    for kk in range(K // _LANES)
                ]
                for k in range(K):
                    a = att_rows[k // _LANES][k % _LANES]
                    for j in range(n_dblk):
                        accs[j] = accs[j] + a * nv[pl.ds(nbase + k * D + j * _LANES, _LANES)]
                for j in range(n_dblk):
                    out_v[pl.ds(obase + j * _LANES, _LANES)] = accs[j]
            pltpu.sync_copy(out_v,
                            out_hbm.at[pl.ds(chunk_base(t) * D, _CHUNK * D)])

        issue(0, 0)

        def outer(it, carry):
            t0 = it * 2
            for b in range(2):
                t = t0 + b

                @pl.when(t + 1 < n_my)
                def _():
                    issue(t + 1, 1 - b)

                @pl.when(t < n_my)
                def _():
                    drain(t, b)
                    compute(t, b)

            return carry

        lax.fori_loop(0, (n_my + 1) // 2, outer, 0)

    return sc_kernel(nbr, att, bias).reshape(N, D)


# speedup vs baseline: 3.1244x; 3.1244x over previous
import functools

import jax
import jax.numpy as jnp
from jax import lax
from jax.experimental import pallas as pl
from jax.experimental.pallas import tpu as pltpu
from jax.experimental.pallas import tpu_sc as plsc

_LANES = 16
_CHUNK = 8


def kernel(nodes, neighbors, attention_scores, bias):
    del nodes
    N, K, _, D = neighbors.shape
    KD = K * D
    nbr = neighbors.reshape(N * K, D)
    att = attention_scores.reshape(N, K)
    assert N % _CHUNK == 0 and D % _LANES == 0 and K % _LANES == 0
    n_chunks = N // _CHUNK
    num_workers = 32
    n_dblk = D // _LANES

    mesh = plsc.VectorSubcoreMesh(core_axis_name="c", subcore_axis_name="s")

    @functools.partial(
        pl.kernel,
        mesh=mesh,
        out_type=jax.ShapeDtypeStruct((N, D), jnp.float32),
        scratch_types=[
            pltpu.VMEM((_CHUNK * K, D), jnp.float32),
            pltpu.VMEM((_CHUNK * K, D), jnp.float32),
            pltpu.VMEM((_CHUNK, K), jnp.float32),
            pltpu.VMEM((_CHUNK, K), jnp.float32),
            pltpu.VMEM((_CHUNK, D), jnp.float32),
            pltpu.VMEM((D,), jnp.float32),
            pltpu.SemaphoreType.DMA,
            pltpu.SemaphoreType.DMA,
            pltpu.SemaphoreType.DMA,
            pltpu.SemaphoreType.DMA,
        ],
    )
    def sc_kernel(nbr_hbm, att_hbm, bias_hbm, out_hbm,
                  nbr_v0, nbr_v1, att_v0, att_v1, out_v, bias_v,
                  sn0, sn1, sa0, sa1):
        nbr_bufs = (nbr_v0, nbr_v1)
        att_bufs = (att_v0, att_v1)
        cid = lax.axis_index("c")
        sid = lax.axis_index("s")
        wid = sid * 2 + cid
        sems_n = (sn0, sn1)
        sems_a = (sa0, sa1)
        pltpu.sync_copy(bias_hbm, bias_v)
        n_my = (n_chunks - wid + num_workers - 1) // num_workers

        def chunk_base(t):
            return (wid + t * num_workers) * _CHUNK

        def issue(t, b):
            base = chunk_base(t)
            pltpu.async_copy(nbr_hbm.at[pl.ds(base * K, _CHUNK * K), :],
                             nbr_bufs[b], sems_n[b])
            pltpu.async_copy(att_hbm.at[pl.ds(base, _CHUNK), :],
                             att_bufs[b], sems_a[b])

        def drain(t, b):
            base = chunk_base(t)
            pltpu.make_async_copy(nbr_hbm.at[pl.ds(base * K, _CHUNK * K), :],
                                  nbr_bufs[b], sems_n[b]).wait()
            pltpu.make_async_copy(att_hbm.at[pl.ds(base, _CHUNK), :],
                                  att_bufs[b], sems_a[b]).wait()

        def compute(t, b):
            nv = nbr_bufs[b]
            av = att_bufs[b]

            def node_body(i, c):
                krow = i * K
                accs = [bias_v[pl.ds(j * _LANES, _LANES)] for j in range(n_dblk)]
                att_rows = [
                    av[i, pl.ds(kk * _LANES, _LANES)]
                    for kk in range(K // _LANES)
                ]
                for k in range(K):
                    a = att_rows[k // _LANES][k % _LANES]
                    row = krow + k
                    for j in range(n_dblk):
                        accs[j] = accs[j] + a * nv[row, pl.ds(j * _LANES, _LANES)]
                for j in range(n_dblk):
                    out_v[i, pl.ds(j * _LANES, _LANES)] = accs[j]
                return c

            lax.fori_loop(0, _CHUNK, node_body, 0)
            pltpu.sync_copy(out_v, out_hbm.at[pl.ds(chunk_base(t), _CHUNK), :])

        issue(0, 0)

        def outer(it, carry):
            t0 = it * 2
            for b in range(2):
                t = t0 + b

                @pl.when(t + 1 < n_my)
                def _():
                    issue(t + 1, 1 - b)

                @pl.when(t < n_my)
                def _():
                    drain(t, b)
                    compute(t, b)

            return carry

        lax.fori_loop(0, (n_my + 1) // 2, outer, 0)

    return sc_kernel(nbr, att, bias)


# speedup vs baseline: 3.1782x; 1.0172x over previous
import functools

import jax
import jax.numpy as jnp
from jax import lax
from jax.experimental import pallas as pl
from jax.experimental.pallas import tpu as pltpu
from jax.experimental.pallas import tpu_sc as plsc

_LANES = 16
_CHUNK = 8


def kernel(nodes, neighbors, attention_scores, bias):
    del nodes
    N, K, _, D = neighbors.shape
    nbr = neighbors.reshape(N * K, D)
    att = attention_scores.reshape(N, K)
    assert N % _CHUNK == 0 and D % _LANES == 0 and K % _LANES == 0
    n_chunks = N // _CHUNK
    num_workers = 32
    n_dblk = D // _LANES

    mesh = plsc.VectorSubcoreMesh(core_axis_name="c", subcore_axis_name="s")

    @functools.partial(
        pl.kernel,
        mesh=mesh,
        out_type=jax.ShapeDtypeStruct((N, D), jnp.float32),
        scratch_types=[
            pltpu.VMEM((_CHUNK * K, D), jnp.float32),
            pltpu.VMEM((_CHUNK * K, D), jnp.float32),
            pltpu.VMEM((_CHUNK, K), jnp.float32),
            pltpu.VMEM((_CHUNK, K), jnp.float32),
            pltpu.VMEM((_CHUNK, D), jnp.float32),
            pltpu.VMEM((_CHUNK, D), jnp.float32),
            pltpu.VMEM((D,), jnp.float32),
            pltpu.SemaphoreType.DMA,
            pltpu.SemaphoreType.DMA,
            pltpu.SemaphoreType.DMA,
            pltpu.SemaphoreType.DMA,
            pltpu.SemaphoreType.DMA,
            pltpu.SemaphoreType.DMA,
        ],
    )
    def sc_kernel(nbr_hbm, att_hbm, bias_hbm, out_hbm,
                  nbr_v0, nbr_v1, att_v0, att_v1, out_v0, out_v1, bias_v,
                  sn0, sn1, sa0, sa1, so0, so1):
        nbr_bufs = (nbr_v0, nbr_v1)
        att_bufs = (att_v0, att_v1)
        out_bufs = (out_v0, out_v1)
        sems_n = (sn0, sn1)
        sems_a = (sa0, sa1)
        sems_o = (so0, so1)
        cid = lax.axis_index("c")
        sid = lax.axis_index("s")
        wid = sid * 2 + cid
        pltpu.sync_copy(bias_hbm, bias_v)
        n_my = (n_chunks - wid + num_workers - 1) // num_workers

        def chunk_base(t):
            return (wid + t * num_workers) * _CHUNK

        def issue(t, b):
            base = chunk_base(t)
            pltpu.async_copy(nbr_hbm.at[pl.ds(base * K, _CHUNK * K), :],
                             nbr_bufs[b], sems_n[b])
            pltpu.async_copy(att_hbm.at[pl.ds(base, _CHUNK), :],
                             att_bufs[b], sems_a[b])

        def drain(t, b):
            base = chunk_base(t)
            pltpu.make_async_copy(nbr_hbm.at[pl.ds(base * K, _CHUNK * K), :],
                                  nbr_bufs[b], sems_n[b]).wait()
            pltpu.make_async_copy(att_hbm.at[pl.ds(base, _CHUNK), :],
                                  att_bufs[b], sems_a[b]).wait()

        def store_wait(t, b):
            pltpu.make_async_copy(out_bufs[b],
                                  out_hbm.at[pl.ds(chunk_base(t), _CHUNK), :],
                                  sems_o[b]).wait()

        def compute(t, b):
            nv = nbr_bufs[b]
            av = att_bufs[b]
            ov = out_bufs[b]

            @pl.when(t >= 2)
            def _():
                store_wait(t - 2, b)

            def node_body(i, c):
                krow = i * K
                accs = [bias_v[pl.ds(j * _LANES, _LANES)] for j in range(n_dblk)]
                att_rows = [
                    av[i, pl.ds(kk * _LANES, _LANES)]
                    for kk in range(K // _LANES)
                ]
                for k in range(K):
                    a = att_rows[k // _LANES][k % _LANES]
                    row = krow + k
                    for j in range(n_dblk):
                        accs[j] = accs[j] + a * nv[row, pl.ds(j * _LANES, _LANES)]
                for j in range(n_dblk):
                    ov[i, pl.ds(j * _LANES, _LANES)] = accs[j]
                return c

            lax.fori_loop(0, _CHUNK, node_body, 0)
            pltpu.async_copy(ov, out_hbm.at[pl.ds(chunk_base(t), _CHUNK), :],
                             sems_o[b])

        issue(0, 0)

        def outer(it, carry):
            t0 = it * 2
            for b in range(2):
                t = t0 + b

                @pl.when(t + 1 < n_my)
                def _():
                    issue(t + 1, 1 - b)

                @pl.when(t < n_my)
                def _():
                    drain(t, b)
                    compute(t, b)

            return carry

        lax.fori_loop(0, (n_my + 1) // 2, outer, 0)

        for b in range(2):
            @pl.when(n_my > b)
            def _(b=b):
                t_last = ((n_my - 1 - b) // 2) * 2 + b
                store_wait(t_last, b)

    return sc_kernel(nbr, att, bias)


# speedup vs baseline: 3.3360x; 1.0496x over previous
import functools

import jax
import jax.numpy as jnp
from jax import lax
from jax.experimental import pallas as pl
from jax.experimental.pallas import tpu as pltpu
from jax.experimental.pallas import tpu_sc as plsc

_LANES = 16
_CHUNK = 8


def kernel(nodes, neighbors, attention_scores, bias):
    del nodes
    N, K, _, D = neighbors.shape
    nbr = neighbors.reshape(N * K, D)
    att = attention_scores.reshape(N, K)
    assert N % _CHUNK == 0 and D % _LANES == 0 and K % _LANES == 0
    n_chunks = N // _CHUNK
    num_workers = 32
    n_dblk = D // _LANES

    mesh = plsc.VectorSubcoreMesh(core_axis_name="c", subcore_axis_name="s")

    @functools.partial(
        pl.kernel,
        mesh=mesh,
        out_type=jax.ShapeDtypeStruct((N, D), jnp.float32),
        scratch_types=[
            pltpu.VMEM((_CHUNK * K, D), jnp.float32),
            pltpu.VMEM((_CHUNK * K, D), jnp.float32),
            pltpu.VMEM((_CHUNK * K, D), jnp.float32),
            pltpu.VMEM((_CHUNK, K), jnp.float32),
            pltpu.VMEM((_CHUNK, K), jnp.float32),
            pltpu.VMEM((_CHUNK, K), jnp.float32),
            pltpu.VMEM((_CHUNK, D), jnp.float32),
            pltpu.VMEM((_CHUNK, D), jnp.float32),
            pltpu.VMEM((_CHUNK, D), jnp.float32),
            pltpu.VMEM((D,), jnp.float32),
        ] + [pltpu.SemaphoreType.DMA] * 9,
    )
    def sc_kernel(nbr_hbm, att_hbm, bias_hbm, out_hbm,
                  nbr_v0, nbr_v1, nbr_v2, att_v0, att_v1, att_v2,
                  out_v0, out_v1, out_v2, bias_v,
                  sn0, sn1, sn2, sa0, sa1, sa2, so0, so1, so2):
        nbr_bufs = (nbr_v0, nbr_v1, nbr_v2)
        att_bufs = (att_v0, att_v1, att_v2)
        out_bufs = (out_v0, out_v1, out_v2)
        sems_n = (sn0, sn1, sn2)
        sems_a = (sa0, sa1, sa2)
        sems_o = (so0, so1, so2)
        cid = lax.axis_index("c")
        sid = lax.axis_index("s")
        wid = sid * 2 + cid
        pltpu.sync_copy(bias_hbm, bias_v)
        n_my = (n_chunks - wid + num_workers - 1) // num_workers

        def chunk_base(t):
            return (wid + t * num_workers) * _CHUNK

        def issue(t, b):
            base = chunk_base(t)
            pltpu.async_copy(nbr_hbm.at[pl.ds(base * K, _CHUNK * K), :],
                             nbr_bufs[b], sems_n[b])
            pltpu.async_copy(att_hbm.at[pl.ds(base, _CHUNK), :],
                             att_bufs[b], sems_a[b])

        def drain(t, b):
            base = chunk_base(t)
            pltpu.make_async_copy(nbr_hbm.at[pl.ds(base * K, _CHUNK * K), :],
                                  nbr_bufs[b], sems_n[b]).wait()
            pltpu.make_async_copy(att_hbm.at[pl.ds(base, _CHUNK), :],
                                  att_bufs[b], sems_a[b]).wait()

        def store_wait(t, b):
            pltpu.make_async_copy(out_bufs[b],
                                  out_hbm.at[pl.ds(chunk_base(t), _CHUNK), :],
                                  sems_o[b]).wait()

        def compute(t, b):
            nv = nbr_bufs[b]
            av = att_bufs[b]
            ov = out_bufs[b]

            @pl.when(t >= 3)
            def _():
                store_wait(t - 3, b)

            def node_body(i, c):
                krow = i * K
                accs = [bias_v[pl.ds(j * _LANES, _LANES)] for j in range(n_dblk)]
                att_rows = [
                    av[i, pl.ds(kk * _LANES, _LANES)]
                    for kk in range(K // _LANES)
                ]
                for k in range(K):
                    a = att_rows[k // _LANES][k % _LANES]
                    row = krow + k
                    for j in range(n_dblk):
                        accs[j] = accs[j] + a * nv[row, pl.ds(j * _LANES, _LANES)]
                for j in range(n_dblk):
                    ov[i, pl.ds(j * _LANES, _LANES)] = accs[j]
                return c

            lax.fori_loop(0, _CHUNK, node_body, 0)
            pltpu.async_copy(ov, out_hbm.at[pl.ds(chunk_base(t), _CHUNK), :],
                             sems_o[b])

        issue(0, 0)

        @pl.when(1 < n_my)
        def _():
            issue(1, 1)

        def outer(it, carry):
            t0 = it * 3
            for b in range(3):
                t = t0 + b

                @pl.when(t + 2 < n_my)
                def _():
                    issue(t + 2, (b + 2) % 3)

                @pl.when(t < n_my)
                def _():
                    drain(t, b)
                    compute(t, b)

            return carry

        lax.fori_loop(0, (n_my + 2) // 3, outer, 0)

        for b in range(3):
            @pl.when(n_my > b)
            def _(b=b):
                t_last = ((n_my - 1 - b) // 3) * 3 + b
                store_wait(t_last, b)

    return sc_kernel(nbr, att, bias)
